# baseline (speedup 1.0000x reference)
import functools

import jax
import jax.numpy as jnp
from jax import lax
from jax.experimental import pallas as pl
from jax.experimental.pallas import tpu as pltpu
from jax.experimental.pallas import tpu_sc as plsc

_N = 65536
_D = 32
_K = 512
_BLK = 16384
_CHUNK = 128

_NC = 2
_NS = 16
_NW = _NC * _NS

_NSPLIT = 2
_NH = _N // _NSPLIT


def _tc_body(x_ref, v_ref, idx2_ref, vt_ref, loss_ref):
    xb = x_ref[...]
    v = v_ref[...]
    xv = jnp.dot(xb, v, preferred_element_type=jnp.float32)
    rownorm = jnp.sum(xb * xb, axis=1, keepdims=True)
    vnorm = jnp.sum(v * v, axis=0, keepdims=True)
    d = (rownorm - 2.0 * xv) + vnorm
    m = jnp.min(d, axis=1, keepdims=True)
    iota = lax.broadcasted_iota(jnp.int32, (1, _K), 1)
    idx = jnp.min(jnp.where(d == m, iota, _K), axis=1)
    idx2_ref[...] = idx.reshape(_BLK // _CHUNK, _CHUNK)

    @pl.when(pl.program_id(0) == 0)
    def _():
        loss_ref[0] = 0.0
        vt_ref[...] = v.T

    loss_ref[0] += jnp.sum(m)


def _tc_part(x, vectors):
    n = x.shape[0]
    grid = n // _BLK
    return pl.pallas_call(
        _tc_body,
        grid=(grid,),
        in_specs=[
            pl.BlockSpec((_BLK, _D), lambda i: (i, 0)),
            pl.BlockSpec((_D, _K), lambda i: (0, 0)),
        ],
        out_specs=[
            pl.BlockSpec((_BLK // _CHUNK, _CHUNK), lambda i: (i, 0)),
            pl.BlockSpec((_K, _D), lambda i: (0, 0)),
            pl.BlockSpec(memory_space=pltpu.SMEM),
        ],
        out_shape=[
            jax.ShapeDtypeStruct((n // _CHUNK, _CHUNK), jnp.int32),
            jax.ShapeDtypeStruct((_K, _D), jnp.float32),
            jax.ShapeDtypeStruct((1,), jnp.float32),
        ],
    )(x, vectors)


def _make_sc_gather(n):
    bpw = n // _NW
    nrow = bpw // _CHUNK
    unroll = 8

    @functools.partial(
        pl.kernel,
        out_type=jax.ShapeDtypeStruct((n // 4, 128), jnp.float32),
        mesh=plsc.VectorSubcoreMesh(core_axis_name="c", subcore_axis_name="s"),
        scratch_types=[
            pltpu.VMEM((_K, _D), jnp.float32),
            pltpu.VMEM((nrow, _CHUNK), jnp.int32),
            pltpu.VMEM((bpw // 4, 128), jnp.float32),
        ],
        compiler_params=pltpu.CompilerParams(
            needs_layout_passes=False, use_tc_tiling_on_sc=False),
    )
    def sc_gather(table_hbm, idx_hbm, out_hbm, table_v, idx_v, out_v):
        wid = lax.axis_index("s") * _NC + lax.axis_index("c")
        pltpu.sync_copy(table_hbm, table_v)
        pltpu.sync_copy(idx_hbm.at[pl.ds(wid * nrow, nrow)], idx_v)
        lane = lax.broadcasted_iota(jnp.int32, (16,), 0)
        lane_hi = lane + 16

        def row_body(c):
            def body(i, _):
                for u in range(unroll):
                    p = i * unroll + u
                    kv = plsc.load_gather(
                        idx_v, [jnp.full((16,), c, jnp.int32),
                                jnp.broadcast_to(p, (16,)).astype(jnp.int32)])
                    lo = plsc.load_gather(table_v, [kv, lane])
                    hi = plsc.load_gather(table_v, [kv, lane_hi])
                    q = c * _CHUNK + p
                    off = 32 * (u % 4)
                    out_v[q // 4, pl.ds(off, 16)] = lo
                    out_v[q // 4, pl.ds(off + 16, 16)] = hi
                return 0

            lax.fori_loop(0, _CHUNK // unroll, body, 0)

        for c in range(nrow):
            row_body(c)
        pltpu.sync_copy(out_v, out_hbm.at[pl.ds(wid * (bpw // 4), bpw // 4)])

    return sc_gather


_sc_gather_full = _make_sc_gather(_N)


def kernel(x, vectors):
    idx2, vt, ls = _tc_part(x, vectors)
    q_r = _sc_gather_full(vt, idx2)
    q = jnp.reshape(q_r, (_N, _D))
    loss = ls[0] / (_N * _D)
    return (q, loss, loss, jnp.reshape(idx2, (_N, 1)))


# speedup vs baseline: 1.0052x; 1.0052x over previous
import functools

import jax
import jax.numpy as jnp
from jax import lax
from jax.experimental import pallas as pl
from jax.experimental.pallas import tpu as pltpu
from jax.experimental.pallas import tpu_sc as plsc

_N = 65536
_D = 32
_K = 512
_BLK = 8192
_CHUNK = 128

_NC = 2
_NS = 16
_NW = _NC * _NS

_NSPLIT = 2
_NH = _N // _NSPLIT


def _tc_body(x_ref, v_ref, idx2_ref, vt_ref, loss_ref):
    xb = x_ref[...]
    v = v_ref[...]
    xv = jnp.dot(xb, v, preferred_element_type=jnp.float32)
    rownorm = jnp.sum(xb * xb, axis=1, keepdims=True)
    vnorm = jnp.sum(v * v, axis=0, keepdims=True)
    d = (rownorm - 2.0 * xv) + vnorm
    m = jnp.min(d, axis=1, keepdims=True)
    iota = lax.broadcasted_iota(jnp.int32, (1, _K), 1)
    idx = jnp.min(jnp.where(d == m, iota, _K), axis=1)
    idx2_ref[...] = idx.reshape(_BLK // _CHUNK, _CHUNK)

    @pl.when(pl.program_id(0) == 0)
    def _():
        loss_ref[0] = 0.0
        vt_ref[...] = v.T

    loss_ref[0] += jnp.sum(m)


def _tc_part(x, vectors):
    n = x.shape[0]
    grid = n // _BLK
    return pl.pallas_call(
        _tc_body,
        grid=(grid,),
        in_specs=[
            pl.BlockSpec((_BLK, _D), lambda i: (i, 0)),
            pl.BlockSpec((_D, _K), lambda i: (0, 0)),
        ],
        out_specs=[
            pl.BlockSpec((_BLK // _CHUNK, _CHUNK), lambda i: (i, 0)),
            pl.BlockSpec((_K, _D), lambda i: (0, 0)),
            pl.BlockSpec(memory_space=pltpu.SMEM),
        ],
        out_shape=[
            jax.ShapeDtypeStruct((n // _CHUNK, _CHUNK), jnp.int32),
            jax.ShapeDtypeStruct((_K, _D), jnp.float32),
            jax.ShapeDtypeStruct((1,), jnp.float32),
        ],
    )(x, vectors)


def _make_sc_gather(n):
    bpw = n // _NW
    nrow = bpw // _CHUNK
    unroll = 8

    @functools.partial(
        pl.kernel,
        out_type=jax.ShapeDtypeStruct((n // 4, 128), jnp.float32),
        mesh=plsc.VectorSubcoreMesh(core_axis_name="c", subcore_axis_name="s"),
        scratch_types=[
            pltpu.VMEM((_K, _D), jnp.float32),
            pltpu.VMEM((nrow, _CHUNK), jnp.int32),
            pltpu.VMEM((bpw // 4, 128), jnp.float32),
        ],
        compiler_params=pltpu.CompilerParams(
            needs_layout_passes=False, use_tc_tiling_on_sc=False),
    )
    def sc_gather(table_hbm, idx_hbm, out_hbm, table_v, idx_v, out_v):
        wid = lax.axis_index("s") * _NC + lax.axis_index("c")
        pltpu.sync_copy(table_hbm, table_v)
        pltpu.sync_copy(idx_hbm.at[pl.ds(wid * nrow, nrow)], idx_v)
        lane = lax.broadcasted_iota(jnp.int32, (16,), 0)
        lane_hi = lane + 16

        def row_body(c):
            def body(i, _):
                for u in range(unroll):
                    p = i * unroll + u
                    kv = plsc.load_gather(
                        idx_v, [jnp.full((16,), c, jnp.int32),
                                jnp.broadcast_to(p, (16,)).astype(jnp.int32)])
                    lo = plsc.load_gather(table_v, [kv, lane])
                    hi = plsc.load_gather(table_v, [kv, lane_hi])
                    q = c * _CHUNK + p
                    off = 32 * (u % 4)
                    out_v[q // 4, pl.ds(off, 16)] = lo
                    out_v[q // 4, pl.ds(off + 16, 16)] = hi
                return 0

            lax.fori_loop(0, _CHUNK // unroll, body, 0)

        for c in range(nrow):
            row_body(c)
        pltpu.sync_copy(out_v, out_hbm.at[pl.ds(wid * (bpw // 4), bpw // 4)])

    return sc_gather


_sc_gather_full = _make_sc_gather(_N)


def kernel(x, vectors):
    idx2, vt, ls = _tc_part(x, vectors)
    q_r = _sc_gather_full(vt, idx2)
    q = jnp.reshape(q_r, (_N, _D))
    loss = ls[0] / (_N * _D)
    return (q, loss, loss, jnp.reshape(idx2, (_N, 1)))
